# Initial kernel scaffold; baseline (speedup 1.0000x reference)
import jax
import jax.numpy as jnp
from jax.experimental import pallas as pl


def kernel(x, Wg, bg, W, b):
    raise NotImplementedError("write your pallas kernel here")


# speedup vs baseline: 2.7798x; 2.7798x over previous
import jax
import jax.numpy as jnp
from jax import lax
from jax.experimental import pallas as pl
from jax.experimental.pallas import tpu as pltpu

TOKENS = 4096
D_IN = 2048
D_HID = 2048
N_EXP = 8
K_TOP = 2

BM_A = 512
BN_C = 512


def _gating_body(x_ref, wg_ref, bg_ref, pt_ref, s2_ref, idx_ref, xbf_ref):
    i = pl.program_id(0)
    xv = x_ref[...]
    logits = jnp.dot(xv, wg_ref[...], preferred_element_type=jnp.float32)
    logits = logits + bg_ref[...]
    m = jnp.max(logits, axis=1, keepdims=True)
    e = jnp.exp(logits - m)
    p = e / jnp.sum(e, axis=1, keepdims=True)

    ei = lax.broadcasted_iota(jnp.int32, (BM_A, N_EXP), 1)
    m1 = jnp.max(p, axis=1, keepdims=True)
    fi = jnp.min(jnp.where(p == m1, ei, N_EXP), axis=1, keepdims=True)
    pm = jnp.where(ei == fi, -jnp.inf, p)
    m2 = jnp.max(pm, axis=1, keepdims=True)
    s2_ref[...] = jnp.concatenate([m1, m2], axis=1)

    pt_ref[...] = p.T
    xbf_ref[...] = xv.astype(jnp.bfloat16)

    @pl.when(i == 0)
    def _():
        si = jnp.min(jnp.where(pm == m2, ei, N_EXP), axis=1, keepdims=True)
        ti = lax.broadcasted_iota(jnp.int32, (BM_A, 1), 0)
        fi0 = jnp.max(jnp.where(ti == 0, fi, -1))
        si0 = jnp.max(jnp.where(ti == 0, si, -1))
        li = lax.broadcasted_iota(jnp.int32, (1, N_EXP), 1)
        idx_ref[...] = jnp.where(li == 0, fi0, jnp.where(li == 1, si0, 0))


def _expert_body(idx_ref, x_ref, w_ref, b_ref, s_ref, o_ref):
    k = pl.program_id(1)
    xb = x_ref[...]
    wb = w_ref[0].astype(jnp.bfloat16)
    dot = jnp.dot(xb, wb, preferred_element_type=jnp.float32)
    sel = (lax.broadcasted_iota(jnp.int32, (K_TOP, 1), 0) == k)
    sk = jnp.dot(s_ref[...], sel.astype(jnp.float32),
                 preferred_element_type=jnp.float32)
    contrib = sk * dot + sk * b_ref[0]

    @pl.when(k == 0)
    def _():
        o_ref[...] = contrib

    @pl.when(k == 1)
    def _():
        o_ref[...] += contrib


def kernel(x, Wg, bg, W, b):
    bg2 = bg.reshape(1, N_EXP)
    b3 = b.reshape(N_EXP, 1, D_HID)

    n_blk = TOKENS // BM_A
    probs_t, s2, idx_row, xbf = pl.pallas_call(
        _gating_body,
        grid=(n_blk,),
        in_specs=[
            pl.BlockSpec((BM_A, D_IN), lambda i: (i, 0)),
            pl.BlockSpec((D_IN, N_EXP), lambda i: (0, 0)),
            pl.BlockSpec((1, N_EXP), lambda i: (0, 0)),
        ],
        out_specs=[
            pl.BlockSpec((N_EXP, BM_A), lambda i: (0, i)),
            pl.BlockSpec((BM_A, K_TOP), lambda i: (i, 0)),
            pl.BlockSpec((1, N_EXP), lambda i: (0, 0)),
            pl.BlockSpec((BM_A, D_IN), lambda i: (i, 0)),
        ],
        out_shape=[
            jax.ShapeDtypeStruct((N_EXP, TOKENS), jnp.float32),
            jax.ShapeDtypeStruct((TOKENS, K_TOP), jnp.float32),
            jax.ShapeDtypeStruct((1, N_EXP), jnp.int32),
            jax.ShapeDtypeStruct((TOKENS, D_IN), jnp.bfloat16),
        ],
        compiler_params=pltpu.CompilerParams(
            dimension_semantics=("arbitrary",)),
    )(x, Wg, bg2)
    del probs_t

    idx2 = idx_row[0, :K_TOP]

    out = pl.pallas_call(
        _expert_body,
        grid_spec=pltpu.PrefetchScalarGridSpec(
            num_scalar_prefetch=1,
            grid=(D_HID // BN_C, K_TOP),
            in_specs=[
                pl.BlockSpec((TOKENS, D_IN), lambda n, k, idx: (0, 0)),
                pl.BlockSpec((1, D_IN, BN_C), lambda n, k, idx: (idx[k], 0, n)),
                pl.BlockSpec((1, 1, BN_C), lambda n, k, idx: (idx[k], 0, n)),
                pl.BlockSpec((TOKENS, K_TOP), lambda n, k, idx: (0, 0)),
            ],
            out_specs=pl.BlockSpec((TOKENS, BN_C), lambda n, k, idx: (0, n)),
        ),
        out_shape=jax.ShapeDtypeStruct((TOKENS, D_HID), jnp.float32),
        compiler_params=pltpu.CompilerParams(
            dimension_semantics=("parallel", "arbitrary")),
    )(idx2, xbf, W, b3, s2)
    return out
